# baseline (speedup 1.0000x reference)
import functools

import jax
import jax.numpy as jnp
from jax import lax
from jax.experimental import pallas as pl
from jax.experimental.pallas import tpu as pltpu
from jax.experimental.pallas import tpu_sc as plsc

RES = 2048
C = 8
N = 1048576
HW = RES * RES

NC = 2
NS = 16
NW = NC * NS
NSPLIT = 4
NH = N // NSPLIT
PER_TILE = NH // NW
CHUNK = 1024
NJ = CHUNK // 128
NCHUNKS = PER_TILE // CHUNK


PIX = HW // NW
PCH = 2048
NPCH = PIX // PCH
NIB = 3


def _sc_relayout(d2):
    mesh = plsc.VectorSubcoreMesh(core_axis_name="c", subcore_axis_name="s")

    @functools.partial(
        pl.kernel,
        mesh=mesh,
        compiler_params=pltpu.CompilerParams(
            needs_layout_passes=False, use_tc_tiling_on_sc=False),
        out_type=jax.ShapeDtypeStruct((HW * C,), jnp.float32),
        scratch_types=[pltpu.VMEM((C * PCH,), jnp.float32)] * NIB
        + [pltpu.VMEM((PCH * C,), jnp.float32)] * 2
        + [pltpu.SemaphoreType.DMA] * NIB
        + [pltpu.SemaphoreType.DMA] * 2,
    )
    def k(d_hbm, t_hbm, *sc):
        inb = sc[:NIB]
        outb = sc[NIB:NIB + 2]
        isem = sc[NIB + 2:2 * NIB + 2]
        osem = sc[2 * NIB + 2:]
        wid = lax.axis_index("s") * NC + lax.axis_index("c")
        tile_base = wid * PIX
        lanes = lax.iota(jnp.int32, 16)
        adr0 = jnp.bitwise_and(lanes, 7) * PCH + lax.shift_right_logical(
            lanes, 3)

        def fire_in(g):
            s = g % NIB
            base = tile_base + g * PCH
            for c in range(C):
                pltpu.async_copy(d_hbm.at[c, pl.ds(base, PCH)],
                                 inb[s].at[pl.ds(c * PCH, PCH)], isem[s])

        def wait_in(g):
            s = g % NIB
            for c in range(C):
                pltpu.make_async_copy(d_hbm.at[0, pl.ds(0, PCH)],
                                      inb[s].at[pl.ds(0, PCH)],
                                      isem[s]).wait()

        for g in range(NIB - 1):
            fire_in(g)
        out_cp = [None, None]
        for g in range(NPCH):
            cur = g % 2
            if g + NIB - 1 < NPCH:
                fire_in(g + NIB - 1)
            wait_in(g)
            if out_cp[cur] is not None:
                out_cp[cur].wait()
            src = inb[g % NIB]
            dst = outb[cur]

            def interleave(kk, adr, src=src, dst=dst):
                v = plsc.load_gather(src, [adr])
                dst[pl.ds(kk * 16, 16)] = v
                return adr + 2

            lax.fori_loop(0, PCH // 2, interleave, adr0, unroll=8)
            base = tile_base + g * PCH
            out_cp[cur] = pltpu.async_copy(
                dst, t_hbm.at[pl.ds(base * C, PCH * C)], osem[cur])
        for cp in out_cp:
            if cp is not None:
                cp.wait()

    return k(d2)


def _sc_grid_sample(fx, fy, table):
    mesh = plsc.VectorSubcoreMesh(core_axis_name="c", subcore_axis_name="s")

    slot_scratch = [
        pltpu.VMEM((CHUNK,), jnp.float32),
        pltpu.VMEM((CHUNK,), jnp.float32),
        pltpu.VMEM((CHUNK,), jnp.int32),
        pltpu.VMEM((CHUNK,), jnp.int32),
        pltpu.VMEM((CHUNK,), jnp.int32),
        pltpu.VMEM((CHUNK,), jnp.int32),
        pltpu.VMEM((CHUNK, C), jnp.float32),
        pltpu.VMEM((CHUNK, C), jnp.float32),
        pltpu.VMEM((CHUNK, C), jnp.float32),
        pltpu.VMEM((CHUNK, C), jnp.float32),
        pltpu.VMEM((CHUNK, C), jnp.float32),
    ]

    @functools.partial(
        pl.kernel,
        mesh=mesh,
        compiler_params=pltpu.CompilerParams(
            needs_layout_passes=False, use_tc_tiling_on_sc=False),
        out_type=jax.ShapeDtypeStruct((NH, C), jnp.float32),
        scratch_types=slot_scratch + slot_scratch + [
            pltpu.SemaphoreType.DMA,
            pltpu.SemaphoreType.DMA,
            pltpu.SemaphoreType.DMA,
            pltpu.SemaphoreType.DMA,
            pltpu.SemaphoreType.DMA,
            pltpu.SemaphoreType.DMA,
        ],
    )
    def k(fx_hbm, fy_hbm, tab_hbm, out_hbm, *sc):
        nslot = len(slot_scratch)
        slots = (sc[:nslot], sc[nslot:2 * nslot])
        isem = (sc[2 * nslot], sc[2 * nslot + 1])
        gsem = (sc[2 * nslot + 2], sc[2 * nslot + 3])
        osem = (sc[2 * nslot + 4], sc[2 * nslot + 5])

        wid = lax.axis_index("s") * NC + lax.axis_index("c")
        tile_base = wid * PER_TILE
        lanes = lax.iota(jnp.int32, 16)
        row_off = lax.shift_right_logical(lanes, 3)
        col_idx = jnp.bitwise_and(lanes, 7)

        def fire_in(g, s):
            base = tile_base + g * CHUNK
            fxv, fyv = slots[s][0], slots[s][1]
            pltpu.async_copy(fx_hbm.at[pl.ds(base, CHUNK)], fxv, isem[s])
            pltpu.async_copy(fy_hbm.at[pl.ds(base, CHUNK)], fyv, isem[s])

        def wait_in(s):
            fxv, fyv = slots[s][0], slots[s][1]
            pltpu.make_async_copy(
                fx_hbm.at[pl.ds(0, CHUNK)], fxv, isem[s]).wait()
            pltpu.make_async_copy(
                fy_hbm.at[pl.ds(0, CHUNK)], fyv, isem[s]).wait()

        def prep(s):
            fxv, fyv = slots[s][0], slots[s][1]
            idxs = slots[s][2:6]

            def body(kk, _):
                o = kk * 16
                ix = fxv[pl.ds(o, 16)]
                iy = fyv[pl.ds(o, 16)]
                ix0 = ix.astype(jnp.int32)
                iy0 = iy.astype(jnp.int32)
                ix1 = jnp.minimum(ix0 + 1, RES - 1)
                iy1 = jnp.minimum(iy0 + 1, RES - 1)
                y0 = iy0 * RES
                y1 = iy1 * RES
                idxs[0][pl.ds(o, 16)] = y0 + ix0
                idxs[1][pl.ds(o, 16)] = y0 + ix1
                idxs[2][pl.ds(o, 16)] = y1 + ix0
                idxs[3][pl.ds(o, 16)] = y1 + ix1
                return 0

            lax.fori_loop(0, CHUNK // 16, body, 0, unroll=2)

        def fire_gather(s):
            idxs = slots[s][2:6]
            rows = slots[s][6:10]

            def body(j, _):
                for t in range(4):
                    pltpu.async_copy(
                        tab_hbm.at[idxs[t].at[pl.ds(j * 128, 128)]],
                        rows[t].at[pl.ds(j * 128, 128)], gsem[s])
                return 0

            lax.fori_loop(0, NJ, body, 0)

        def wait_gather(s):
            rows = slots[s][6:10]
            for t in range(4):
                pltpu.make_async_copy(
                    tab_hbm.at[pl.ds(0, CHUNK)], rows[t], gsem[s]).wait()

        def blend(s):
            fxv, fyv = slots[s][0], slots[s][1]
            r00, r01, r10, r11, out_v = slots[s][6:11]

            def body(p, rid):
                v00 = plsc.load_gather(r00, [rid, col_idx])
                v01 = plsc.load_gather(r01, [rid, col_idx])
                v10 = plsc.load_gather(r10, [rid, col_idx])
                v11 = plsc.load_gather(r11, [rid, col_idx])
                bfx = plsc.load_gather(fxv, [rid])
                bfy = plsc.load_gather(fyv, [rid])
                wx = bfx - bfx.astype(jnp.int32).astype(jnp.float32)
                wy = bfy - bfy.astype(jnp.int32).astype(jnp.float32)
                ux = 1.0 - wx
                uy = 1.0 - wy
                acc = (v00 * (ux * uy) + v01 * (wx * uy)
                       + v10 * (ux * wy) + v11 * (wx * wy))
                plsc.store_scatter(out_v, [rid, col_idx], acc)
                return rid + 2

            lax.fori_loop(0, CHUNK // 2, body, row_off, unroll=2)

        def fire_out(g, s):
            base = tile_base + g * CHUNK
            pltpu.async_copy(slots[s][10], out_hbm.at[pl.ds(base, CHUNK)],
                             osem[s])

        def wait_out(s):
            pltpu.make_async_copy(
                slots[s][10], out_hbm.at[pl.ds(0, CHUNK)], osem[s]).wait()

        fire_in(0, 0)
        for g in range(NCHUNKS):
            s = g % 2
            wait_in(s)
            prep(s)
            fire_gather(s)
            if g >= 1:
                ps = (g - 1) % 2
                wait_gather(ps)
                if g >= 3:
                    wait_out(ps)
                blend(ps)
                fire_out(g - 1, ps)
            if g + 1 < NCHUNKS:
                fire_in(g + 1, (g + 1) % 2)
        ls = (NCHUNKS - 1) % 2
        wait_gather(ls)
        wait_out(ls)
        blend(ls)
        fire_out(NCHUNKS - 1, ls)
        wait_out(0)
        wait_out(1)

    return k(fx, fy, table)


def kernel(x, data):
    xs = jax.nn.sigmoid(x)
    xs = xs * 2.0 - 1.0
    gx = xs[:, 1]
    gy = xs[:, 0]
    fx = jnp.clip((gx + 1.0) * 0.5 * (RES - 1), 0.0, float(RES - 1))
    fy = jnp.clip((gy + 1.0) * 0.5 * (RES - 1), 0.0, float(RES - 1))
    table = _sc_relayout(data.reshape(C, HW)).reshape(HW, C)
    parts = [
        _sc_grid_sample(fx[i * NH:(i + 1) * NH], fy[i * NH:(i + 1) * NH],
                        table)
        for i in range(NSPLIT)
    ]
    return jnp.concatenate(parts, axis=0)


# speedup vs baseline: 1.2205x; 1.2205x over previous
import functools

import jax
import jax.numpy as jnp
from jax import lax
from jax.experimental import pallas as pl
from jax.experimental.pallas import tpu as pltpu
from jax.experimental.pallas import tpu_sc as plsc

RES = 2048
C = 8
N = 1048576
HW = RES * RES

NC = 2
NS = 16
NW = NC * NS
NSPLIT = 4
NH = N // NSPLIT
PER_TILE = NH // NW
CHUNK = 1024
NJ = CHUNK // 128
NCHUNKS = PER_TILE // CHUNK


PIX = HW // NW
PCH = 2048
NPCH = PIX // PCH
NIB = 3


def _sc_relayout(d2):
    mesh = plsc.VectorSubcoreMesh(core_axis_name="c", subcore_axis_name="s")

    @functools.partial(
        pl.kernel,
        mesh=mesh,
        compiler_params=pltpu.CompilerParams(
            needs_layout_passes=False, use_tc_tiling_on_sc=False),
        out_type=jax.ShapeDtypeStruct((HW * C,), jnp.float32),
        scratch_types=[pltpu.VMEM((C * (PCH + 8),), jnp.float32)] * NIB
        + [pltpu.VMEM((PCH * C,), jnp.float32)] * 2
        + [pltpu.SemaphoreType.DMA] * NIB
        + [pltpu.SemaphoreType.DMA] * 2,
    )
    def k(d_hbm, t_hbm, *sc):
        inb = sc[:NIB]
        outb = sc[NIB:NIB + 2]
        isem = sc[NIB + 2:2 * NIB + 2]
        osem = sc[2 * NIB + 2:]
        wid = lax.axis_index("s") * NC + lax.axis_index("c")
        tile_base = wid * PIX
        lanes = lax.iota(jnp.int32, 16)
        adr0 = jnp.bitwise_and(lanes, 7) * (PCH + 8) + lax.shift_right_logical(
            lanes, 3)

        def fire_in(g):
            s = g % NIB
            base = tile_base + g * PCH
            for c in range(C):
                pltpu.async_copy(d_hbm.at[c, pl.ds(base, PCH)],
                                 inb[s].at[pl.ds(c * (PCH + 8), PCH)],
                                 isem[s])

        def wait_in(g):
            s = g % NIB
            for c in range(C):
                pltpu.make_async_copy(d_hbm.at[0, pl.ds(0, PCH)],
                                      inb[s].at[pl.ds(0, PCH)],
                                      isem[s]).wait()

        for g in range(NIB - 1):
            fire_in(g)
        out_cp = [None, None]
        for g in range(NPCH):
            cur = g % 2
            if g + NIB - 1 < NPCH:
                fire_in(g + NIB - 1)
            wait_in(g)
            if out_cp[cur] is not None:
                out_cp[cur].wait()
            src = inb[g % NIB]
            dst = outb[cur]

            def interleave(kk, adr, src=src, dst=dst):
                v = plsc.load_gather(src, [adr])
                dst[pl.ds(kk * 16, 16)] = v
                return adr + 2

            lax.fori_loop(0, PCH // 2, interleave, adr0, unroll=8)
            base = tile_base + g * PCH
            out_cp[cur] = pltpu.async_copy(
                dst, t_hbm.at[pl.ds(base * C, PCH * C)], osem[cur])
        for cp in out_cp:
            if cp is not None:
                cp.wait()

    return k(d2)


def _sc_grid_sample(fx, fy, table):
    mesh = plsc.VectorSubcoreMesh(core_axis_name="c", subcore_axis_name="s")

    slot_scratch = [
        pltpu.VMEM((CHUNK,), jnp.float32),
        pltpu.VMEM((CHUNK,), jnp.float32),
        pltpu.VMEM((CHUNK,), jnp.int32),
        pltpu.VMEM((CHUNK,), jnp.int32),
        pltpu.VMEM((CHUNK,), jnp.int32),
        pltpu.VMEM((CHUNK,), jnp.int32),
        pltpu.VMEM((CHUNK, C), jnp.float32),
        pltpu.VMEM((CHUNK, C), jnp.float32),
        pltpu.VMEM((CHUNK, C), jnp.float32),
        pltpu.VMEM((CHUNK, C), jnp.float32),
        pltpu.VMEM((CHUNK, C), jnp.float32),
    ]

    @functools.partial(
        pl.kernel,
        mesh=mesh,
        compiler_params=pltpu.CompilerParams(
            needs_layout_passes=False, use_tc_tiling_on_sc=False),
        out_type=jax.ShapeDtypeStruct((NH, C), jnp.float32),
        scratch_types=slot_scratch + slot_scratch + [
            pltpu.SemaphoreType.DMA,
            pltpu.SemaphoreType.DMA,
            pltpu.SemaphoreType.DMA,
            pltpu.SemaphoreType.DMA,
            pltpu.SemaphoreType.DMA,
            pltpu.SemaphoreType.DMA,
        ],
    )
    def k(fx_hbm, fy_hbm, tab_hbm, out_hbm, *sc):
        nslot = len(slot_scratch)
        slots = (sc[:nslot], sc[nslot:2 * nslot])
        isem = (sc[2 * nslot], sc[2 * nslot + 1])
        gsem = (sc[2 * nslot + 2], sc[2 * nslot + 3])
        osem = (sc[2 * nslot + 4], sc[2 * nslot + 5])

        wid = lax.axis_index("s") * NC + lax.axis_index("c")
        tile_base = wid * PER_TILE
        lanes = lax.iota(jnp.int32, 16)
        row_off = lax.shift_right_logical(lanes, 3)
        col_idx = jnp.bitwise_and(lanes, 7)

        def fire_in(g, s):
            base = tile_base + g * CHUNK
            fxv, fyv = slots[s][0], slots[s][1]
            pltpu.async_copy(fx_hbm.at[pl.ds(base, CHUNK)], fxv, isem[s])
            pltpu.async_copy(fy_hbm.at[pl.ds(base, CHUNK)], fyv, isem[s])

        def wait_in(s):
            fxv, fyv = slots[s][0], slots[s][1]
            pltpu.make_async_copy(
                fx_hbm.at[pl.ds(0, CHUNK)], fxv, isem[s]).wait()
            pltpu.make_async_copy(
                fy_hbm.at[pl.ds(0, CHUNK)], fyv, isem[s]).wait()

        def prep(s):
            fxv, fyv = slots[s][0], slots[s][1]
            idxs = slots[s][2:6]

            def body(kk, _):
                o = kk * 16
                ix = fxv[pl.ds(o, 16)]
                iy = fyv[pl.ds(o, 16)]
                ix0 = ix.astype(jnp.int32)
                iy0 = iy.astype(jnp.int32)
                ix1 = jnp.minimum(ix0 + 1, RES - 1)
                iy1 = jnp.minimum(iy0 + 1, RES - 1)
                y0 = iy0 * RES
                y1 = iy1 * RES
                idxs[0][pl.ds(o, 16)] = y0 + ix0
                idxs[1][pl.ds(o, 16)] = y0 + ix1
                idxs[2][pl.ds(o, 16)] = y1 + ix0
                idxs[3][pl.ds(o, 16)] = y1 + ix1
                return 0

            lax.fori_loop(0, CHUNK // 16, body, 0, unroll=2)

        def fire_gather(s):
            idxs = slots[s][2:6]
            rows = slots[s][6:10]

            def body(j, _):
                for t in range(4):
                    pltpu.async_copy(
                        tab_hbm.at[idxs[t].at[pl.ds(j * 128, 128)]],
                        rows[t].at[pl.ds(j * 128, 128)], gsem[s])
                return 0

            lax.fori_loop(0, NJ, body, 0)

        def wait_gather(s):
            rows = slots[s][6:10]
            for t in range(4):
                pltpu.make_async_copy(
                    tab_hbm.at[pl.ds(0, CHUNK)], rows[t], gsem[s]).wait()

        def blend(s):
            fxv, fyv = slots[s][0], slots[s][1]
            r00, r01, r10, r11, out_v = slots[s][6:11]

            def body(p, rid):
                v00 = plsc.load_gather(r00, [rid, col_idx])
                v01 = plsc.load_gather(r01, [rid, col_idx])
                v10 = plsc.load_gather(r10, [rid, col_idx])
                v11 = plsc.load_gather(r11, [rid, col_idx])
                bfx = plsc.load_gather(fxv, [rid])
                bfy = plsc.load_gather(fyv, [rid])
                wx = bfx - bfx.astype(jnp.int32).astype(jnp.float32)
                wy = bfy - bfy.astype(jnp.int32).astype(jnp.float32)
                ux = 1.0 - wx
                uy = 1.0 - wy
                acc = (v00 * (ux * uy) + v01 * (wx * uy)
                       + v10 * (ux * wy) + v11 * (wx * wy))
                plsc.store_scatter(out_v, [rid, col_idx], acc)
                return rid + 2

            lax.fori_loop(0, CHUNK // 2, body, row_off, unroll=2)

        def fire_out(g, s):
            base = tile_base + g * CHUNK
            pltpu.async_copy(slots[s][10], out_hbm.at[pl.ds(base, CHUNK)],
                             osem[s])

        def wait_out(s):
            pltpu.make_async_copy(
                slots[s][10], out_hbm.at[pl.ds(0, CHUNK)], osem[s]).wait()

        fire_in(0, 0)
        for g in range(NCHUNKS):
            s = g % 2
            wait_in(s)
            prep(s)
            fire_gather(s)
            if g >= 1:
                ps = (g - 1) % 2
                wait_gather(ps)
                if g >= 3:
                    wait_out(ps)
                blend(ps)
                fire_out(g - 1, ps)
            if g + 1 < NCHUNKS:
                fire_in(g + 1, (g + 1) % 2)
        ls = (NCHUNKS - 1) % 2
        wait_gather(ls)
        wait_out(ls)
        blend(ls)
        fire_out(NCHUNKS - 1, ls)
        wait_out(0)
        wait_out(1)

    return k(fx, fy, table)


def kernel(x, data):
    xs = jax.nn.sigmoid(x)
    xs = xs * 2.0 - 1.0
    gx = xs[:, 1]
    gy = xs[:, 0]
    fx = jnp.clip((gx + 1.0) * 0.5 * (RES - 1), 0.0, float(RES - 1))
    fy = jnp.clip((gy + 1.0) * 0.5 * (RES - 1), 0.0, float(RES - 1))
    table = _sc_relayout(data.reshape(C, HW)).reshape(HW, C)
    parts = [
        _sc_grid_sample(fx[i * NH:(i + 1) * NH], fy[i * NH:(i + 1) * NH],
                        table)
        for i in range(NSPLIT)
    ]
    return jnp.concatenate(parts, axis=0)


# speedup vs baseline: 1.2997x; 1.0649x over previous
import functools

import jax
import jax.numpy as jnp
from jax import lax
from jax.experimental import pallas as pl
from jax.experimental.pallas import tpu as pltpu
from jax.experimental.pallas import tpu_sc as plsc

RES = 2048
C = 8
N = 1048576
HW = RES * RES

NC = 2
NS = 16
NW = NC * NS
NSPLIT = 4
NH = N // NSPLIT
PER_TILE = NH // NW
CHUNK = 1024
NJ = CHUNK // 128
NCHUNKS = PER_TILE // CHUNK


PIX = HW // NW
PCH = 2048
NPCH = PIX // PCH
NIB = 3


def _sc_relayout(d2):
    mesh = plsc.VectorSubcoreMesh(core_axis_name="c", subcore_axis_name="s")

    @functools.partial(
        pl.kernel,
        mesh=mesh,
        compiler_params=pltpu.CompilerParams(
            needs_layout_passes=False, use_tc_tiling_on_sc=False),
        out_type=jax.ShapeDtypeStruct((HW * C,), jnp.float32),
        scratch_types=[pltpu.VMEM((C * (PCH + 8),), jnp.float32)] * NIB
        + [pltpu.VMEM((PCH * C,), jnp.float32)] * 2
        + [pltpu.SemaphoreType.DMA] * NIB
        + [pltpu.SemaphoreType.DMA] * 2,
    )
    def k(d_hbm, t_hbm, *sc):
        inb = sc[:NIB]
        outb = sc[NIB:NIB + 2]
        isem = sc[NIB + 2:2 * NIB + 2]
        osem = sc[2 * NIB + 2:]
        wid = lax.axis_index("s") * NC + lax.axis_index("c")
        tile_base = wid * PIX
        lanes = lax.iota(jnp.int32, 16)
        adr0 = jnp.bitwise_and(lanes, 7) * (PCH + 8) + lax.shift_right_logical(
            lanes, 3)

        def fire_in(g):
            s = g % NIB
            base = tile_base + g * PCH
            for c in range(C):
                pltpu.async_copy(d_hbm.at[c, pl.ds(base, PCH)],
                                 inb[s].at[pl.ds(c * (PCH + 8), PCH)],
                                 isem[s])

        def wait_in(g):
            s = g % NIB
            for c in range(C):
                pltpu.make_async_copy(d_hbm.at[0, pl.ds(0, PCH)],
                                      inb[s].at[pl.ds(0, PCH)],
                                      isem[s]).wait()

        for g in range(NIB - 1):
            fire_in(g)
        out_cp = [None, None]
        for g in range(NPCH):
            cur = g % 2
            if g + NIB - 1 < NPCH:
                fire_in(g + NIB - 1)
            wait_in(g)
            if out_cp[cur] is not None:
                out_cp[cur].wait()
            src = inb[g % NIB]
            dst = outb[cur]

            def interleave(kk, adr, src=src, dst=dst):
                v = plsc.load_gather(src, [adr])
                dst[pl.ds(kk * 16, 16)] = v
                return adr + 2

            lax.fori_loop(0, PCH // 2, interleave, adr0, unroll=8)
            base = tile_base + g * PCH
            out_cp[cur] = pltpu.async_copy(
                dst, t_hbm.at[pl.ds(base * C, PCH * C)], osem[cur])
        for cp in out_cp:
            if cp is not None:
                cp.wait()

    return k(d2)


def _sc_grid_sample(fx, fy, table):
    mesh = plsc.VectorSubcoreMesh(core_axis_name="c", subcore_axis_name="s")

    slot_scratch = [
        pltpu.VMEM((CHUNK,), jnp.float32),
        pltpu.VMEM((CHUNK,), jnp.float32),
        pltpu.VMEM((CHUNK,), jnp.int32),
        pltpu.VMEM((CHUNK,), jnp.int32),
        pltpu.VMEM((CHUNK,), jnp.int32),
        pltpu.VMEM((CHUNK,), jnp.int32),
        pltpu.VMEM((CHUNK, C), jnp.float32),
        pltpu.VMEM((CHUNK, C), jnp.float32),
        pltpu.VMEM((CHUNK, C), jnp.float32),
        pltpu.VMEM((CHUNK, C), jnp.float32),
        pltpu.VMEM((CHUNK, C), jnp.float32),
    ]

    @functools.partial(
        pl.kernel,
        mesh=mesh,
        compiler_params=pltpu.CompilerParams(
            needs_layout_passes=False, use_tc_tiling_on_sc=False),
        out_type=jax.ShapeDtypeStruct((NH, C), jnp.float32),
        scratch_types=slot_scratch + slot_scratch + [
            pltpu.SemaphoreType.DMA,
            pltpu.SemaphoreType.DMA,
            pltpu.SemaphoreType.DMA,
            pltpu.SemaphoreType.DMA,
            pltpu.SemaphoreType.DMA,
            pltpu.SemaphoreType.DMA,
        ],
    )
    def k(fx_hbm, fy_hbm, tab_hbm, out_hbm, *sc):
        nslot = len(slot_scratch)
        slots = (sc[:nslot], sc[nslot:2 * nslot])
        isem = (sc[2 * nslot], sc[2 * nslot + 1])
        gsem = (sc[2 * nslot + 2], sc[2 * nslot + 3])
        osem = (sc[2 * nslot + 4], sc[2 * nslot + 5])

        wid = lax.axis_index("s") * NC + lax.axis_index("c")
        tile_base = wid * PER_TILE
        lanes = lax.iota(jnp.int32, 16)
        row_off = lax.shift_right_logical(lanes, 3)
        col_idx = jnp.bitwise_and(lanes, 7)

        def fire_in(g, s):
            base = tile_base + g * CHUNK
            fxv, fyv = slots[s][0], slots[s][1]
            pltpu.async_copy(fx_hbm.at[pl.ds(base, CHUNK)], fxv, isem[s])
            pltpu.async_copy(fy_hbm.at[pl.ds(base, CHUNK)], fyv, isem[s])

        def wait_in(s):
            fxv, fyv = slots[s][0], slots[s][1]
            pltpu.make_async_copy(
                fx_hbm.at[pl.ds(0, CHUNK)], fxv, isem[s]).wait()
            pltpu.make_async_copy(
                fy_hbm.at[pl.ds(0, CHUNK)], fyv, isem[s]).wait()

        def prep(s):
            fxv, fyv = slots[s][0], slots[s][1]
            idxs = slots[s][2:6]

            def body(kk, _):
                o = kk * 16
                ix = fxv[pl.ds(o, 16)]
                iy = fyv[pl.ds(o, 16)]
                ix0 = ix.astype(jnp.int32)
                iy0 = iy.astype(jnp.int32)
                ix1 = jnp.minimum(ix0 + 1, RES - 1)
                iy1 = jnp.minimum(iy0 + 1, RES - 1)
                y0 = iy0 * RES
                y1 = iy1 * RES
                idxs[0][pl.ds(o, 16)] = y0 + ix0
                idxs[1][pl.ds(o, 16)] = y0 + ix1
                idxs[2][pl.ds(o, 16)] = y1 + ix0
                idxs[3][pl.ds(o, 16)] = y1 + ix1
                return 0

            lax.fori_loop(0, CHUNK // 16, body, 0, unroll=2)

        def fire_gather(s):
            idxs = slots[s][2:6]
            rows = slots[s][6:10]

            def body(j, _):
                for t in range(4):
                    pltpu.async_copy(
                        tab_hbm.at[idxs[t].at[pl.ds(j * 128, 128)]],
                        rows[t].at[pl.ds(j * 128, 128)], gsem[s])
                return 0

            lax.fori_loop(0, NJ, body, 0)

        def wait_gather(s):
            rows = slots[s][6:10]
            for t in range(4):
                pltpu.make_async_copy(
                    tab_hbm.at[pl.ds(0, CHUNK)], rows[t], gsem[s]).wait()

        sel = [row_off + 2 * j for j in range(8)]

        def blend(s):
            fxv, fyv = slots[s][0], slots[s][1]
            r00, r01, r10, r11, out_v = slots[s][6:11]

            def body(kk, rid):
                o = kk * 16
                cx = fxv[pl.ds(o, 16)]
                cy = fyv[pl.ds(o, 16)]
                for j in range(8):
                    v00 = plsc.load_gather(r00, [rid, col_idx])
                    v01 = plsc.load_gather(r01, [rid, col_idx])
                    v10 = plsc.load_gather(r10, [rid, col_idx])
                    v11 = plsc.load_gather(r11, [rid, col_idx])
                    bfx = jnp.take_along_axis(cx, sel[j], axis=0)
                    bfy = jnp.take_along_axis(cy, sel[j], axis=0)
                    wx = bfx - bfx.astype(jnp.int32).astype(jnp.float32)
                    wy = bfy - bfy.astype(jnp.int32).astype(jnp.float32)
                    ux = 1.0 - wx
                    uy = 1.0 - wy
                    acc = (v00 * (ux * uy) + v01 * (wx * uy)
                           + v10 * (ux * wy) + v11 * (wx * wy))
                    plsc.store_scatter(out_v, [rid, col_idx], acc)
                    rid = rid + 2
                return rid

            lax.fori_loop(0, CHUNK // 16, body, row_off)

        def fire_out(g, s):
            base = tile_base + g * CHUNK
            pltpu.async_copy(slots[s][10], out_hbm.at[pl.ds(base, CHUNK)],
                             osem[s])

        def wait_out(s):
            pltpu.make_async_copy(
                slots[s][10], out_hbm.at[pl.ds(0, CHUNK)], osem[s]).wait()

        fire_in(0, 0)
        for g in range(NCHUNKS):
            s = g % 2
            wait_in(s)
            prep(s)
            fire_gather(s)
            if g >= 1:
                ps = (g - 1) % 2
                wait_gather(ps)
                if g >= 3:
                    wait_out(ps)
                blend(ps)
                fire_out(g - 1, ps)
            if g + 1 < NCHUNKS:
                fire_in(g + 1, (g + 1) % 2)
        ls = (NCHUNKS - 1) % 2
        wait_gather(ls)
        wait_out(ls)
        blend(ls)
        fire_out(NCHUNKS - 1, ls)
        wait_out(0)
        wait_out(1)

    return k(fx, fy, table)


def kernel(x, data):
    xs = jax.nn.sigmoid(x)
    xs = xs * 2.0 - 1.0
    gx = xs[:, 1]
    gy = xs[:, 0]
    fx = jnp.clip((gx + 1.0) * 0.5 * (RES - 1), 0.0, float(RES - 1))
    fy = jnp.clip((gy + 1.0) * 0.5 * (RES - 1), 0.0, float(RES - 1))
    table = _sc_relayout(data.reshape(C, HW)).reshape(HW, C)
    parts = [
        _sc_grid_sample(fx[i * NH:(i + 1) * NH], fy[i * NH:(i + 1) * NH],
                        table)
        for i in range(NSPLIT)
    ]
    return jnp.concatenate(parts, axis=0)
